# baseline (speedup 1.0000x reference)
import functools

import jax
import jax.numpy as jnp
from jax import lax
from jax.experimental import pallas as pl
from jax.experimental.pallas import tpu as pltpu
from jax.experimental.pallas import tpu_sc as plsc

H = 64
B = 16384
NC, NS = 2, 16
NW = NC * NS
BPW = B // NW
CHUNK = 128
NCHUNK = BPW // CHUNK


def _gather_body(idx_hbm, table_hbm, out_hbm, idx_v, rows_v, sem):
  wid = lax.axis_index("s") * NC + lax.axis_index("c")
  base = wid * BPW
  pltpu.sync_copy(idx_hbm.at[wid], idx_v)
  copies = [
      pltpu.async_copy(
          table_hbm.at[idx_v.at[j]],
          rows_v.at[pl.ds(j * CHUNK, CHUNK)],
          sem,
      )
      for j in range(NCHUNK)
  ]
  for c in copies:
    c.wait()
  pltpu.sync_copy(rows_v, out_hbm.at[pl.ds(base, BPW)])


@jax.jit
def _gather(idx, table):
  mesh = plsc.VectorSubcoreMesh(core_axis_name="c", subcore_axis_name="s")
  return pl.kernel(
      _gather_body,
      out_type=jax.ShapeDtypeStruct((B, H), jnp.bfloat16),
      mesh=mesh,
      scratch_types=[
          pltpu.VMEM((NCHUNK, CHUNK), jnp.int32),
          pltpu.VMEM((BPW, H), jnp.bfloat16),
          pltpu.SemaphoreType.DMA,
      ],
      compiler_params=pltpu.CompilerParams(use_tc_tiling_on_sc=False),
  )(idx, table)


NSTATES = 1000000
CB = 8192


def _convert_body(xt_ref, o_ref):
  o_ref[...] = xt_ref[...].T.astype(jnp.bfloat16)


@jax.jit
def _convert(table_t):
  return pl.pallas_call(
      _convert_body,
      grid=(pl.cdiv(NSTATES, CB),),
      in_specs=[pl.BlockSpec((H, CB), lambda i: (0, i))],
      out_specs=pl.BlockSpec((CB, H), lambda i: (i, 0)),
      out_shape=jax.ShapeDtypeStruct((NSTATES, H), jnp.bfloat16),
  )(table_t)


RB = 2048


def _mlp_body(f_ref, w1_ref, b1_ref, w2t_ref, b2_ref, o_ref):
  x = f_ref[...].astype(jnp.float32)
  h = jnp.dot(x, w1_ref[...], preferred_element_type=jnp.float32)
  h = jnp.maximum(h + b1_ref[...], 0.0)
  o_ref[...] = (
      jnp.sum(h * w2t_ref[...], axis=1, keepdims=True) + b2_ref[...]
  )


@jax.jit
def _mlp(feats, w1, b1r, w2t, b2r):
  return pl.pallas_call(
      _mlp_body,
      grid=(B // RB,),
      in_specs=[
          pl.BlockSpec((RB, H), lambda i: (i, 0)),
          pl.BlockSpec((H, H), lambda i: (0, 0)),
          pl.BlockSpec((1, H), lambda i: (0, 0)),
          pl.BlockSpec((1, H), lambda i: (0, 0)),
          pl.BlockSpec((1, 1), lambda i: (0, 0)),
      ],
      out_specs=pl.BlockSpec((RB, 1), lambda i: (i, 0)),
      out_shape=jax.ShapeDtypeStruct((B, 1), jnp.float32),
  )(feats, w1, b1r, w2t, b2r)


def kernel(state_indices, embedding, W1, b1, W2, b2):
  idx = state_indices.astype(jnp.int32).reshape(NW, NCHUNK, CHUNK)
  table = _convert(embedding.T)
  feats = _gather(idx, table)
  out = _mlp(
      feats,
      W1,
      b1.reshape(1, H),
      W2.reshape(1, H),
      b2.reshape(1, 1),
  )
  return out.reshape(B)


# speedup vs baseline: 1.1386x; 1.1386x over previous
import functools

import jax
import jax.numpy as jnp
from jax import lax
from jax.experimental import pallas as pl
from jax.experimental.pallas import tpu as pltpu
from jax.experimental.pallas import tpu_sc as plsc

H = 64
B = 16384
NC, NS = 2, 16
NW = NC * NS
BPW = B // NW
CHUNK = 128
NCHUNK = BPW // CHUNK


def _gather_body(idx_hbm, table_hbm, out_hbm, idx_v, rows_v, sem):
  wid = lax.axis_index("s") * NC + lax.axis_index("c")
  base = wid * BPW
  pltpu.sync_copy(idx_hbm.at[wid], idx_v)
  copies = [
      pltpu.async_copy(
          table_hbm.at[idx_v.at[j]],
          rows_v.at[pl.ds(j * CHUNK, CHUNK)],
          sem,
      )
      for j in range(NCHUNK)
  ]
  for c in copies:
    c.wait()
  pltpu.sync_copy(rows_v, out_hbm.at[pl.ds(base, BPW)])


@jax.jit
def _gather(idx, table):
  mesh = plsc.VectorSubcoreMesh(core_axis_name="c", subcore_axis_name="s")
  return pl.kernel(
      _gather_body,
      out_type=jax.ShapeDtypeStruct((B, H), jnp.float32),
      mesh=mesh,
      scratch_types=[
          pltpu.VMEM((NCHUNK, CHUNK), jnp.int32),
          pltpu.VMEM((BPW, H), jnp.float32),
          pltpu.SemaphoreType.DMA,
      ],
      compiler_params=pltpu.CompilerParams(use_tc_tiling_on_sc=False),
  )(idx, table)


NSTATES = 1000000
CB = 8192


def _convert_body(xt_ref, o_ref):
  o_ref[...] = xt_ref[...].T


@jax.jit
def _convert(table_t):
  return pl.pallas_call(
      _convert_body,
      grid=(pl.cdiv(NSTATES, CB),),
      in_specs=[pl.BlockSpec((H, CB), lambda i: (0, i))],
      out_specs=pl.BlockSpec((CB, H), lambda i: (i, 0)),
      out_shape=jax.ShapeDtypeStruct((NSTATES, H), jnp.float32),
  )(table_t)


RB = 2048


def _mlp_body(f_ref, w1_ref, b1_ref, w2t_ref, b2_ref, o_ref):
  h = jnp.dot(f_ref[...], w1_ref[...], preferred_element_type=jnp.float32)
  h = jnp.maximum(h + b1_ref[...], 0.0)
  o_ref[...] = (
      jnp.sum(h * w2t_ref[...], axis=1, keepdims=True) + b2_ref[...]
  )


@jax.jit
def _mlp(feats, w1, b1r, w2t, b2r):
  return pl.pallas_call(
      _mlp_body,
      grid=(B // RB,),
      in_specs=[
          pl.BlockSpec((RB, H), lambda i: (i, 0)),
          pl.BlockSpec((H, H), lambda i: (0, 0)),
          pl.BlockSpec((1, H), lambda i: (0, 0)),
          pl.BlockSpec((1, H), lambda i: (0, 0)),
          pl.BlockSpec((1, 1), lambda i: (0, 0)),
      ],
      out_specs=pl.BlockSpec((RB, 1), lambda i: (i, 0)),
      out_shape=jax.ShapeDtypeStruct((B, 1), jnp.float32),
  )(feats, w1, b1r, w2t, b2r)


def kernel(state_indices, embedding, W1, b1, W2, b2):
  idx = state_indices.astype(jnp.int32).reshape(NW, NCHUNK, CHUNK)
  table = _convert(embedding.T)
  feats = _gather(idx, table)
  out = _mlp(
      feats,
      W1,
      b1.reshape(1, H),
      W2.reshape(1, H),
      b2.reshape(1, 1),
  )
  return out.reshape(B)


# speedup vs baseline: 2.5940x; 2.2783x over previous
import functools

import jax
import jax.numpy as jnp
from jax import lax
from jax.experimental import pallas as pl
from jax.experimental.pallas import tpu as pltpu
from jax.experimental.pallas import tpu_sc as plsc

H = 64
B = 16384
NC, NS = 2, 16
NW = NC * NS
BPW = B // NW
CHUNK = 128
NCHUNK = BPW // CHUNK


def _gather_body(idx_hbm, table_hbm, out_hbm, idx_v, rows_v, sem):
  wid = lax.axis_index("s") * NC + lax.axis_index("c")
  base = wid * BPW
  pltpu.sync_copy(idx_hbm.at[wid], idx_v)
  copies = [
      pltpu.async_copy(
          table_hbm.at[idx_v.at[j]],
          rows_v.at[pl.ds(j * CHUNK, CHUNK)],
          sem,
      )
      for j in range(NCHUNK)
  ]
  for c in copies:
    c.wait()
  pltpu.sync_copy(rows_v, out_hbm.at[pl.ds(base, BPW)])


@jax.jit
def _gather(idx, table):
  mesh = plsc.VectorSubcoreMesh(core_axis_name="c", subcore_axis_name="s")
  return pl.kernel(
      _gather_body,
      out_type=jax.ShapeDtypeStruct((B, 2 * H), jnp.float32),
      mesh=mesh,
      scratch_types=[
          pltpu.VMEM((NCHUNK, CHUNK), jnp.int32),
          pltpu.VMEM((BPW, 2 * H), jnp.float32),
          pltpu.SemaphoreType.DMA,
      ],
      compiler_params=pltpu.CompilerParams(use_tc_tiling_on_sc=False),
  )(idx, table)


NSTATES = 1000000
CB = 8192


def _convert_body(xt_ref, o_ref):
  x = xt_ref[...]
  for j in range(CB // 256):
    o_ref[pl.ds(j * 128, 128), 0:H] = x[:, 256 * j:256 * j + 128].T
    o_ref[pl.ds(j * 128, 128), H:2 * H] = (
        x[:, 256 * j + 128:256 * j + 256].T
    )


@jax.jit
def _convert(table_t):
  return pl.pallas_call(
      _convert_body,
      grid=(pl.cdiv(NSTATES, CB),),
      in_specs=[pl.BlockSpec((H, CB), lambda i: (0, i))],
      out_specs=pl.BlockSpec((CB // 2, 2 * H), lambda i: (i, 0)),
      out_shape=jax.ShapeDtypeStruct((NSTATES // 2, 2 * H), jnp.float32),
  )(table_t)


RB = 2048


def _mlp_body(f_ref, par_ref, w1_ref, b1_ref, w2t_ref, b2_ref, o_ref):
  f = f_ref[...]
  x = jnp.where(par_ref[...] > 0, f[:, H:], f[:, :H])
  h = jnp.dot(x, w1_ref[...], preferred_element_type=jnp.float32)
  h = jnp.maximum(h + b1_ref[...], 0.0)
  o_ref[...] = (
      jnp.sum(h * w2t_ref[...], axis=1, keepdims=True) + b2_ref[...]
  )


@jax.jit
def _mlp(feats, par, w1, b1r, w2t, b2r):
  return pl.pallas_call(
      _mlp_body,
      grid=(B // RB,),
      in_specs=[
          pl.BlockSpec((RB, 2 * H), lambda i: (i, 0)),
          pl.BlockSpec((RB, 1), lambda i: (i, 0)),
          pl.BlockSpec((H, H), lambda i: (0, 0)),
          pl.BlockSpec((1, H), lambda i: (0, 0)),
          pl.BlockSpec((1, H), lambda i: (0, 0)),
          pl.BlockSpec((1, 1), lambda i: (0, 0)),
      ],
      out_specs=pl.BlockSpec((RB, 1), lambda i: (i, 0)),
      out_shape=jax.ShapeDtypeStruct((B, 1), jnp.float32),
  )(feats, par, w1, b1r, w2t, b2r)


def kernel(state_indices, embedding, W1, b1, W2, b2):
  idx = state_indices.astype(jnp.int32)
  idx_pair = ((idx // 256) * 128 + (idx & 127)).reshape(NW, NCHUNK, CHUNK)
  par = ((idx >> 7) & 1).reshape(B, 1)
  table = _convert(embedding.T)
  feats = _gather(idx_pair, table)
  out = _mlp(
      feats,
      par,
      W1,
      b1.reshape(1, H),
      W2.reshape(1, H),
      b2.reshape(1, 1),
  )
  return out.reshape(B)


# speedup vs baseline: 2.5976x; 1.0014x over previous
import jax
import jax.numpy as jnp
from jax import lax
from jax.experimental import pallas as pl
from jax.experimental.pallas import tpu as pltpu
from jax.experimental.pallas import tpu_sc as plsc

H = 64
B = 16384
NC, NS = 2, 16
NW = NC * NS
BPW = B // NW
CHUNK = 128
NCHUNK = BPW // CHUNK


def _gather_body(idx_hbm, table_hbm, out_hbm, idx_v, rows_v, sem):
  wid = lax.axis_index("s") * NC + lax.axis_index("c")
  base = wid * BPW
  pltpu.sync_copy(idx_hbm.at[wid], idx_v)
  copies = [
      pltpu.async_copy(
          table_hbm.at[idx_v.at[j]],
          rows_v.at[pl.ds(j * CHUNK, CHUNK)],
          sem,
      )
      for j in range(NCHUNK)
  ]
  for c in copies:
    c.wait()
  pltpu.sync_copy(rows_v, out_hbm.at[pl.ds(base, BPW)])


@jax.jit
def _gather(idx, table):
  mesh = plsc.VectorSubcoreMesh(core_axis_name="c", subcore_axis_name="s")
  return pl.kernel(
      _gather_body,
      out_type=jax.ShapeDtypeStruct((B, 2 * H), jnp.float32),
      mesh=mesh,
      scratch_types=[
          pltpu.VMEM((NCHUNK, CHUNK), jnp.int32),
          pltpu.VMEM((BPW, 2 * H), jnp.float32),
          pltpu.SemaphoreType.DMA,
      ],
      compiler_params=pltpu.CompilerParams(use_tc_tiling_on_sc=False),
  )(idx, table)


NSTATES = 1000000
CB = 8192


def _convert_body(xt_ref, o_ref):
  x = xt_ref[...]
  for j in range(CB // 256):
    o_ref[pl.ds(j * 128, 128), 0:H] = x[:, 256 * j:256 * j + 128].T
    o_ref[pl.ds(j * 128, 128), H:2 * H] = (
        x[:, 256 * j + 128:256 * j + 256].T
    )


@jax.jit
def _convert(table_t):
  return pl.pallas_call(
      _convert_body,
      grid=(pl.cdiv(NSTATES, CB),),
      in_specs=[pl.BlockSpec((H, CB), lambda i: (0, i))],
      out_specs=pl.BlockSpec((CB // 2, 2 * H), lambda i: (i, 0)),
      out_shape=jax.ShapeDtypeStruct((NSTATES // 2, 2 * H), jnp.float32),
  )(table_t)


RB = 2048


def _mlp_body(f_ref, par_ref, w1_ref, b1_ref, w2t_ref, b2_ref, o_ref):
  f = f_ref[...]
  x = jnp.where(par_ref[...] > 0, f[:, H:], f[:, :H])
  h = jnp.dot(x, w1_ref[...], preferred_element_type=jnp.float32)
  h = jnp.maximum(h + b1_ref[...], 0.0)
  o_ref[...] = (
      jnp.sum(h * w2t_ref[...], axis=1, keepdims=True) + b2_ref[...]
  )


@jax.jit
def _mlp(feats, par, w1, b1r, w2t, b2r):
  return pl.pallas_call(
      _mlp_body,
      grid=(B // RB,),
      in_specs=[
          pl.BlockSpec((RB, 2 * H), lambda i: (i, 0)),
          pl.BlockSpec((RB, 1), lambda i: (i, 0)),
          pl.BlockSpec((H, H), lambda i: (0, 0)),
          pl.BlockSpec((1, H), lambda i: (0, 0)),
          pl.BlockSpec((1, H), lambda i: (0, 0)),
          pl.BlockSpec((1, 1), lambda i: (0, 0)),
      ],
      out_specs=pl.BlockSpec((RB, 1), lambda i: (i, 0)),
      out_shape=jax.ShapeDtypeStruct((B, 1), jnp.float32),
  )(feats, par, w1, b1r, w2t, b2r)


def kernel(state_indices, embedding, W1, b1, W2, b2):
  idx = state_indices.astype(jnp.int32)
  idx_pair = ((idx // 256) * 128 + (idx & 127)).reshape(NW, NCHUNK, CHUNK)
  par = ((idx >> 7) & 1).reshape(B, 1)
  table = _convert(embedding.T)
  feats = _gather(idx_pair, table)
  out = _mlp(
      feats,
      par,
      W1,
      b1.reshape(1, H),
      W2.reshape(1, H),
      b2.reshape(1, 1),
  )
  return out.reshape(B)
